# baseline (speedup 1.0000x reference)
import functools

import jax
import jax.numpy as jnp
from jax import lax
from jax.experimental import pallas as pl
from jax.experimental.pallas import tpu as pltpu
from jax.experimental.pallas import tpu_sc as plsc

N_NODES = 10000
N_EDGES = 320000
N_PAD = 10240
NUM_CORES = 2
NUM_SUBCORES = 16
NW = NUM_CORES * NUM_SUBCORES
CHUNK64, NCH64, NBUF64 = 128, 160, 5
CHUNK16, NCH16, NBUF16 = 128, 80, 4
ROWS_PER_SUB = N_PAD // NUM_SUBCORES
BLK = 1280
GRID = N_PAD // BLK


def _pipelined_chunk_loop(xp_ref, idx_s, idx_d, buf, acc, sem_g, sem_s,
                          n_chunks, nbuf):
  for b in range(nbuf):
    pltpu.async_copy(xp_ref.at[idx_s.at[b]], buf.at[b], sem_g[b])

  def step(g, carry):
    for b in range(nbuf):
      j = g * nbuf + b
      pltpu.make_async_copy(xp_ref.at[idx_s.at[0]],
                            buf.at[b], sem_g[b]).wait()
      pltpu.async_copy(buf.at[b], acc.at[idx_d.at[j]], sem_s[b], add=True)
    for b in range(nbuf):
      j = g * nbuf + b
      pltpu.make_async_copy(buf.at[b], acc.at[idx_d.at[0]],
                            sem_s[b]).wait()
      jn = jnp.minimum(j + nbuf, n_chunks - 1)
      pltpu.async_copy(xp_ref.at[idx_s.at[jn]], buf.at[b], sem_g[b])
    return carry

  lax.fori_loop(0, n_chunks // nbuf, step, 0)
  for b in range(nbuf):
    pltpu.make_async_copy(xp_ref.at[idx_s.at[0]],
                          buf.at[b], sem_g[b]).wait()


def _make_feat_agg(width, chunk, n_chunks, nbuf):
  mesh = plsc.VectorSubcoreMesh(core_axis_name="c", subcore_axis_name="s")

  def body(xp_hbm, src_hbm, dst_hbm, zeros_hbm, y_hbm,
           idx_s, idx_d, buf, acc, *sems):
    sem_g = sems[:nbuf]
    sem_s = sems[nbuf:]
    c = lax.axis_index("c")
    s = lax.axis_index("s")
    r0 = s * ROWS_PER_SUB
    pltpu.sync_copy(zeros_hbm, acc.at[pl.ds(r0, ROWS_PER_SUB)])
    pltpu.sync_copy(src_hbm.at[s], idx_s)
    pltpu.sync_copy(dst_hbm.at[s], idx_d)
    plsc.subcore_barrier()
    _pipelined_chunk_loop(xp_hbm.at[c], idx_s, idx_d, buf, acc,
                          sem_g, sem_s, n_chunks, nbuf)
    plsc.subcore_barrier()
    pltpu.sync_copy(acc.at[pl.ds(r0, ROWS_PER_SUB)],
                    y_hbm.at[c, pl.ds(r0, ROWS_PER_SUB)])

  return pl.kernel(
      body,
      out_type=jax.ShapeDtypeStruct((NUM_CORES, N_PAD, width), jnp.float32),
      mesh=mesh,
      compiler_params=pltpu.CompilerParams(use_tc_tiling_on_sc=False),
      scratch_types=[
          pltpu.VMEM((n_chunks, chunk), jnp.int32),
          pltpu.VMEM((n_chunks, chunk), jnp.int32),
          pltpu.VMEM((nbuf, chunk, width), jnp.float32),
          pltpu.VMEM_SHARED((N_PAD, width), jnp.float32),
      ] + [pltpu.SemaphoreType.DMA] * (2 * nbuf),
  )


def _make_edge_agg(width, chunk, n_chunks, nbuf):
  mesh = plsc.VectorSubcoreMesh(core_axis_name="c", subcore_axis_name="s")

  def body(xp_hbm, src_hbm, dst_hbm, zeros_hbm, y_hbm,
           idx_s, idx_d, buf, acc, *sems):
    sem_g = sems[:nbuf]
    sem_s = sems[nbuf:]
    c = lax.axis_index("c")
    s = lax.axis_index("s")
    widx = c * NUM_SUBCORES + s
    r0 = s * ROWS_PER_SUB
    pltpu.sync_copy(zeros_hbm, acc.at[pl.ds(r0, ROWS_PER_SUB)])
    pltpu.sync_copy(src_hbm.at[widx], idx_s)
    pltpu.sync_copy(dst_hbm.at[widx], idx_d)
    plsc.subcore_barrier()
    _pipelined_chunk_loop(xp_hbm, idx_s, idx_d, buf, acc,
                          sem_g, sem_s, n_chunks, nbuf)
    plsc.subcore_barrier()
    pltpu.sync_copy(acc.at[pl.ds(r0, ROWS_PER_SUB)],
                    y_hbm.at[c, pl.ds(r0, ROWS_PER_SUB)])

  return pl.kernel(
      body,
      out_type=jax.ShapeDtypeStruct((NUM_CORES, N_PAD, width), jnp.float32),
      mesh=mesh,
      compiler_params=pltpu.CompilerParams(use_tc_tiling_on_sc=False),
      scratch_types=[
          pltpu.VMEM((n_chunks, chunk), jnp.int32),
          pltpu.VMEM((n_chunks, chunk), jnp.int32),
          pltpu.VMEM((nbuf, chunk, width), jnp.float32),
          pltpu.VMEM_SHARED((N_PAD, width), jnp.float32),
      ] + [pltpu.SemaphoreType.DMA] * (2 * nbuf),
  )


_agg64 = _make_feat_agg(64, CHUNK64, NCH64, NBUF64)
_agg16 = _make_edge_agg(16, CHUNK16, NCH16, NBUF16)


def _chunked_edges(idx, chunk, n_chunks, n_shards):
  e_pad = n_shards * chunk * n_chunks
  fill = jnp.full((e_pad - N_EDGES,), N_NODES, jnp.int32)
  return jnp.concatenate([idx, fill]).reshape(n_shards, n_chunks, chunk)


def _prep_body(h_ref, w_ref, d0_ref, d1_ref, xp_ref, dinv_ref):
  deg = d0_ref[:, 0:1] + d1_ref[:, 0:1] + 1.0
  dinv = lax.rsqrt(deg)
  hl = jnp.dot(h_ref[...], w_ref[...], preferred_element_type=jnp.float32)
  xp_ref[...] = dinv * hl
  dinv_ref[...] = jnp.broadcast_to(dinv, dinv_ref.shape)


def _tc_prep(h0, w0, d0, d1):
  return pl.pallas_call(
      _prep_body,
      grid=(GRID,),
      in_specs=[
          pl.BlockSpec((BLK, 128), lambda i: (i, 0)),
          pl.BlockSpec((128, 128), lambda i: (0, 0)),
          pl.BlockSpec((BLK, 16), lambda i: (i, 0)),
          pl.BlockSpec((BLK, 16), lambda i: (i, 0)),
      ],
      out_specs=[
          pl.BlockSpec((BLK, 128), lambda i: (i, 0)),
          pl.BlockSpec((BLK, 128), lambda i: (i, 0)),
      ],
      out_shape=[jax.ShapeDtypeStruct((N_PAD, 128), jnp.float32)] * 2,
  )(h0, w0, d0, d1)


def _bound_body(y_ref, xp_ref, dinv_ref, b_ref, w_ref, out_ref):
  t = dinv_ref[...] * (y_ref[...] + xp_ref[...]) + b_ref[...]
  h = jnp.maximum(t, 0.0)
  hl = jnp.dot(h, w_ref[...], preferred_element_type=jnp.float32)
  out_ref[...] = dinv_ref[...] * hl


def _tc_boundary(y, xp, dinv, b, w):
  return pl.pallas_call(
      _bound_body,
      grid=(GRID,),
      in_specs=[
          pl.BlockSpec((BLK, 128), lambda i: (i, 0)),
          pl.BlockSpec((BLK, 128), lambda i: (i, 0)),
          pl.BlockSpec((BLK, 128), lambda i: (i, 0)),
          pl.BlockSpec((1, 128), lambda i: (0, 0)),
          pl.BlockSpec((128, 128), lambda i: (0, 0)),
      ],
      out_specs=pl.BlockSpec((BLK, 128), lambda i: (i, 0)),
      out_shape=jax.ShapeDtypeStruct((N_PAD, 128), jnp.float32),
  )(y, xp, dinv, b, w)


def _final_body(y0_ref, y1_ref, xp_ref, dinv_ref, b_ref, out_ref):
  out_ref[...] = (dinv_ref[...] * (y0_ref[...] + y1_ref[...] + xp_ref[...])
                  + b_ref[...])


def _tc_final(y0, y1, xp, dinv, b):
  return pl.pallas_call(
      _final_body,
      grid=(GRID,),
      in_specs=[
          pl.BlockSpec((BLK, 128), lambda i: (i, 0)),
          pl.BlockSpec((BLK, 128), lambda i: (i, 0)),
          pl.BlockSpec((BLK, 128), lambda i: (i, 0)),
          pl.BlockSpec((BLK, 128), lambda i: (i, 0)),
          pl.BlockSpec((1, 128), lambda i: (0, 0)),
      ],
      out_specs=pl.BlockSpec((BLK, 128), lambda i: (i, 0)),
      out_shape=jax.ShapeDtypeStruct((N_PAD, 128), jnp.float32),
  )(y0, y1, xp, dinv, b)


def kernel(x, saf, dsdf, edge_index, aoa,
           W0, W1, W2, W3, W4, W5, b0, b1, b2, b3, b4, b5):
  del aoa
  f32 = jnp.float32

  h0 = jnp.concatenate([x, saf, dsdf], axis=1)
  h0 = jnp.pad(h0, ((0, N_PAD - N_NODES), (0, 0)))

  src = edge_index[0]
  dst = edge_index[1]
  srcs64 = _chunked_edges(src, CHUNK64, NCH64, NUM_SUBCORES)
  dsts64 = _chunked_edges(dst, CHUNK64, NCH64, NUM_SUBCORES)
  srcs16 = _chunked_edges(src, CHUNK16, NCH16, NW)
  dsts16 = _chunked_edges(dst, CHUNK16, NCH16, NW)

  zeros16 = jnp.zeros((ROWS_PER_SUB, 16), f32)
  zeros64 = jnp.zeros((ROWS_PER_SUB, 64), f32)
  ones16 = jnp.ones((N_PAD, 16), f32)

  degp = _agg16(ones16, srcs16, dsts16, zeros16)
  d0 = degp[0]
  d1 = degp[1]

  xp, dinv = _tc_prep(h0, W0, d0, d1)

  ws = [W1, W2, W3, W4, jnp.pad(W5, ((0, 0), (0, 128 - 4)))]
  bs = [b0, b1, b2, b3, b4]
  for i in range(5):
    xp3 = jnp.stack([xp[:, :64], xp[:, 64:]])
    yp = _agg64(xp3, srcs64, dsts64, zeros64)
    y = jnp.concatenate([yp[0], yp[1]], axis=1)
    xp = _tc_boundary(y, xp, dinv, bs[i].reshape(1, 128), ws[i])

  xp16 = xp[:, :16]
  yp = _agg16(xp16, srcs16, dsts16, zeros16)
  y0 = jnp.pad(yp[0], ((0, 0), (0, 112)))
  y1 = jnp.pad(yp[1], ((0, 0), (0, 112)))
  b5p = jnp.pad(b5, (0, 128 - 4)).reshape(1, 128)
  out = _tc_final(y0, y1, xp, dinv, b5p)
  return out[:N_NODES, :4]


# speedup vs baseline: 1.4923x; 1.4923x over previous
import functools

import jax
import jax.numpy as jnp
from jax import lax
from jax.experimental import pallas as pl
from jax.experimental.pallas import tpu as pltpu
from jax.experimental.pallas import tpu_sc as plsc

N_NODES = 10000
N_EDGES = 320000
N_PAD = 10240
NUM_CORES = 2
NUM_SUBCORES = 16
NW = NUM_CORES * NUM_SUBCORES
CHUNK64, NCH64, NBUF64 = 240, 84, 3
CHUNK16, NCH16, NBUF16 = 128, 80, 4
ROWS_PER_SUB = N_PAD // NUM_SUBCORES
BLK = 1280
GRID = N_PAD // BLK


def _pipelined_chunk_loop(xp_ref, idx_s, idx_d, buf, acc, sem_g, sem_s,
                          n_chunks, nbuf):
  for b in range(nbuf):
    pltpu.async_copy(xp_ref.at[idx_s.at[b]], buf.at[b], sem_g[b])

  def step(g, carry):
    for b in range(nbuf):
      j = g * nbuf + b
      pltpu.make_async_copy(xp_ref.at[idx_s.at[0]],
                            buf.at[b], sem_g[b]).wait()
      pltpu.async_copy(buf.at[b], acc.at[idx_d.at[j]], sem_s[b], add=True)
      pltpu.make_async_copy(buf.at[b], acc.at[idx_d.at[0]],
                            sem_s[b]).wait()
      jn = jnp.minimum(j + nbuf, n_chunks - 1)
      pltpu.async_copy(xp_ref.at[idx_s.at[jn]], buf.at[b], sem_g[b])
    return carry

  lax.fori_loop(0, n_chunks // nbuf, step, 0)
  for b in range(nbuf):
    pltpu.make_async_copy(xp_ref.at[idx_s.at[0]],
                          buf.at[b], sem_g[b]).wait()


def _make_feat_agg(width, chunk, n_chunks, nbuf):
  mesh = plsc.VectorSubcoreMesh(core_axis_name="c", subcore_axis_name="s")

  def body(xp_hbm, src_hbm, dst_hbm, zeros_hbm, y_hbm,
           idx_s, idx_d, buf, acc, *sems):
    sem_g = sems[:nbuf]
    sem_s = sems[nbuf:]
    c = lax.axis_index("c")
    s = lax.axis_index("s")
    r0 = s * ROWS_PER_SUB
    pltpu.sync_copy(zeros_hbm, acc.at[pl.ds(r0, ROWS_PER_SUB)])
    pltpu.sync_copy(src_hbm.at[s], idx_s)
    pltpu.sync_copy(dst_hbm.at[s], idx_d)
    plsc.subcore_barrier()
    _pipelined_chunk_loop(xp_hbm.at[c], idx_s, idx_d, buf, acc,
                          sem_g, sem_s, n_chunks, nbuf)
    plsc.subcore_barrier()
    pltpu.sync_copy(acc.at[pl.ds(r0, ROWS_PER_SUB)],
                    y_hbm.at[c, pl.ds(r0, ROWS_PER_SUB)])

  return pl.kernel(
      body,
      out_type=jax.ShapeDtypeStruct((NUM_CORES, N_PAD, width), jnp.float32),
      mesh=mesh,
      compiler_params=pltpu.CompilerParams(use_tc_tiling_on_sc=False),
      scratch_types=[
          pltpu.VMEM((n_chunks, chunk), jnp.int32),
          pltpu.VMEM((n_chunks, chunk), jnp.int32),
          pltpu.VMEM((nbuf, chunk, width), jnp.float32),
          pltpu.VMEM_SHARED((N_PAD, width), jnp.float32),
      ] + [pltpu.SemaphoreType.DMA] * (2 * nbuf),
  )


def _make_edge_agg(width, chunk, n_chunks, nbuf):
  mesh = plsc.VectorSubcoreMesh(core_axis_name="c", subcore_axis_name="s")

  def body(xp_hbm, src_hbm, dst_hbm, zeros_hbm, y_hbm,
           idx_s, idx_d, buf, acc, *sems):
    sem_g = sems[:nbuf]
    sem_s = sems[nbuf:]
    c = lax.axis_index("c")
    s = lax.axis_index("s")
    widx = c * NUM_SUBCORES + s
    r0 = s * ROWS_PER_SUB
    pltpu.sync_copy(zeros_hbm, acc.at[pl.ds(r0, ROWS_PER_SUB)])
    pltpu.sync_copy(src_hbm.at[widx], idx_s)
    pltpu.sync_copy(dst_hbm.at[widx], idx_d)
    plsc.subcore_barrier()
    _pipelined_chunk_loop(xp_hbm, idx_s, idx_d, buf, acc,
                          sem_g, sem_s, n_chunks, nbuf)
    plsc.subcore_barrier()
    pltpu.sync_copy(acc.at[pl.ds(r0, ROWS_PER_SUB)],
                    y_hbm.at[c, pl.ds(r0, ROWS_PER_SUB)])

  return pl.kernel(
      body,
      out_type=jax.ShapeDtypeStruct((NUM_CORES, N_PAD, width), jnp.float32),
      mesh=mesh,
      compiler_params=pltpu.CompilerParams(use_tc_tiling_on_sc=False),
      scratch_types=[
          pltpu.VMEM((n_chunks, chunk), jnp.int32),
          pltpu.VMEM((n_chunks, chunk), jnp.int32),
          pltpu.VMEM((nbuf, chunk, width), jnp.float32),
          pltpu.VMEM_SHARED((N_PAD, width), jnp.float32),
      ] + [pltpu.SemaphoreType.DMA] * (2 * nbuf),
  )


_agg64 = _make_feat_agg(64, CHUNK64, NCH64, NBUF64)
_agg16 = _make_edge_agg(16, CHUNK16, NCH16, NBUF16)


def _chunked_edges(idx, chunk, n_chunks, n_shards):
  e_pad = n_shards * chunk * n_chunks
  fill = jnp.full((e_pad - N_EDGES,), N_NODES, jnp.int32)
  return jnp.concatenate([idx, fill]).reshape(n_shards, n_chunks, chunk)


def _prep_body(h_ref, w_ref, d0_ref, d1_ref, xp_ref, dinv_ref):
  deg = d0_ref[:, 0:1] + d1_ref[:, 0:1] + 1.0
  dinv = lax.rsqrt(deg)
  hl = jnp.dot(h_ref[...], w_ref[...], preferred_element_type=jnp.float32)
  xp_ref[...] = dinv * hl
  dinv_ref[...] = jnp.broadcast_to(dinv, dinv_ref.shape)


def _tc_prep(h0, w0, d0, d1):
  return pl.pallas_call(
      _prep_body,
      grid=(GRID,),
      in_specs=[
          pl.BlockSpec((BLK, 128), lambda i: (i, 0)),
          pl.BlockSpec((128, 128), lambda i: (0, 0)),
          pl.BlockSpec((BLK, 16), lambda i: (i, 0)),
          pl.BlockSpec((BLK, 16), lambda i: (i, 0)),
      ],
      out_specs=[
          pl.BlockSpec((BLK, 128), lambda i: (i, 0)),
          pl.BlockSpec((BLK, 128), lambda i: (i, 0)),
      ],
      out_shape=[jax.ShapeDtypeStruct((N_PAD, 128), jnp.float32)] * 2,
  )(h0, w0, d0, d1)


def _bound_body(y_ref, xp_ref, dinv_ref, b_ref, w_ref, out_ref):
  t = dinv_ref[...] * (y_ref[...] + xp_ref[...]) + b_ref[...]
  h = jnp.maximum(t, 0.0)
  hl = jnp.dot(h, w_ref[...], preferred_element_type=jnp.float32)
  out_ref[...] = dinv_ref[...] * hl


def _tc_boundary(y, xp, dinv, b, w):
  return pl.pallas_call(
      _bound_body,
      grid=(GRID,),
      in_specs=[
          pl.BlockSpec((BLK, 128), lambda i: (i, 0)),
          pl.BlockSpec((BLK, 128), lambda i: (i, 0)),
          pl.BlockSpec((BLK, 128), lambda i: (i, 0)),
          pl.BlockSpec((1, 128), lambda i: (0, 0)),
          pl.BlockSpec((128, 128), lambda i: (0, 0)),
      ],
      out_specs=pl.BlockSpec((BLK, 128), lambda i: (i, 0)),
      out_shape=jax.ShapeDtypeStruct((N_PAD, 128), jnp.float32),
  )(y, xp, dinv, b, w)


def _final_body(y0_ref, y1_ref, xp_ref, dinv_ref, b_ref, out_ref):
  out_ref[...] = (dinv_ref[...] * (y0_ref[...] + y1_ref[...] + xp_ref[...])
                  + b_ref[...])


def _tc_final(y0, y1, xp, dinv, b):
  return pl.pallas_call(
      _final_body,
      grid=(GRID,),
      in_specs=[
          pl.BlockSpec((BLK, 128), lambda i: (i, 0)),
          pl.BlockSpec((BLK, 128), lambda i: (i, 0)),
          pl.BlockSpec((BLK, 128), lambda i: (i, 0)),
          pl.BlockSpec((BLK, 128), lambda i: (i, 0)),
          pl.BlockSpec((1, 128), lambda i: (0, 0)),
      ],
      out_specs=pl.BlockSpec((BLK, 128), lambda i: (i, 0)),
      out_shape=jax.ShapeDtypeStruct((N_PAD, 128), jnp.float32),
  )(y0, y1, xp, dinv, b)


def kernel(x, saf, dsdf, edge_index, aoa,
           W0, W1, W2, W3, W4, W5, b0, b1, b2, b3, b4, b5):
  del aoa
  f32 = jnp.float32

  h0 = jnp.concatenate([x, saf, dsdf], axis=1)
  h0 = jnp.pad(h0, ((0, N_PAD - N_NODES), (0, 0)))

  src = edge_index[0]
  dst = edge_index[1]
  srcs64 = _chunked_edges(src, CHUNK64, NCH64, NUM_SUBCORES)
  dsts64 = _chunked_edges(dst, CHUNK64, NCH64, NUM_SUBCORES)
  srcs16 = _chunked_edges(src, CHUNK16, NCH16, NW)
  dsts16 = _chunked_edges(dst, CHUNK16, NCH16, NW)

  zeros16 = jnp.zeros((ROWS_PER_SUB, 16), f32)
  zeros64 = jnp.zeros((ROWS_PER_SUB, 64), f32)
  ones16 = jnp.ones((N_PAD, 16), f32)

  degp = _agg16(ones16, srcs16, dsts16, zeros16)
  d0 = degp[0]
  d1 = degp[1]

  xp, dinv = _tc_prep(h0, W0, d0, d1)

  ws = [W1, W2, W3, W4, jnp.pad(W5, ((0, 0), (0, 128 - 4)))]
  bs = [b0, b1, b2, b3, b4]
  for i in range(5):
    xp3 = jnp.stack([xp[:, :64], xp[:, 64:]])
    yp = _agg64(xp3, srcs64, dsts64, zeros64)
    y = jnp.concatenate([yp[0], yp[1]], axis=1)
    xp = _tc_boundary(y, xp, dinv, bs[i].reshape(1, 128), ws[i])

  xp16 = xp[:, :16]
  yp = _agg16(xp16, srcs16, dsts16, zeros16)
  y0 = jnp.pad(yp[0], ((0, 0), (0, 112)))
  y1 = jnp.pad(yp[1], ((0, 0), (0, 112)))
  b5p = jnp.pad(b5, (0, 128 - 4)).reshape(1, 128)
  out = _tc_final(y0, y1, xp, dinv, b5p)
  return out[:N_NODES, :4]
